# baseline (speedup 1.0000x reference)
import functools

import jax
import jax.numpy as jnp
from jax import lax
from jax.experimental import pallas as pl
from jax.experimental.pallas import tpu as pltpu
from jax.experimental.pallas import tpu_sc as plsc

NC = 2
NS = 16
NW = NC * NS
CH = 128
LANES = 16


def _degree_kernel(n_pad, n_chunks_per_worker):
  ept = n_pad // NS
  ones_len = ((CH + LANES - 1) // LANES) * LANES
  mesh = plsc.VectorSubcoreMesh(core_axis_name="c", subcore_axis_name="s")

  def body(dst_hbm, out_hbm, dstv, ones_v, zv, acc):
    c = lax.axis_index("c")
    s = lax.axis_index("s")
    wid = s * NC + c
    one16 = jnp.ones((LANES,), jnp.float32)
    z16 = jnp.zeros((LANES,), jnp.float32)
    for j in range(ones_len // LANES):
      ones_v[pl.ds(j * LANES, LANES)] = one16
    for j in range(ept // LANES):
      zv[pl.ds(j * LANES, LANES)] = z16
    pltpu.sync_copy(zv, acc.at[pl.ds(s * ept, ept)])
    ib = wid * n_chunks_per_worker
    pltpu.sync_copy(dst_hbm.at[pl.ds(ib, n_chunks_per_worker), :], dstv)
    plsc.subcore_barrier()

    def chunk(j, carry):
      pltpu.sync_copy(ones_v.at[pl.ds(0, CH)], acc.at[dstv.at[j]], add=True)
      return carry

    lax.fori_loop(0, n_chunks_per_worker, chunk, 0)
    plsc.subcore_barrier()
    pltpu.sync_copy(acc.at[pl.ds(s * ept, ept)],
                    out_hbm.at[pl.ds(c * n_pad + s * ept, ept)])

  return pl.kernel(
      body,
      out_type=jax.ShapeDtypeStruct((NC * n_pad,), jnp.float32),
      mesh=mesh,
      scratch_types=[
          pltpu.VMEM((n_chunks_per_worker, CH), jnp.int32),
          pltpu.VMEM((ones_len,), jnp.float32),
          pltpu.VMEM((ept,), jnp.float32),
          pltpu.VMEM_SHARED((n_pad,), jnp.float32),
      ],
  )


SB = 8


def _edge_agg_kernel(n_acc, d, ncw):
  rpt = n_acc // NS
  zr = 64
  zrep = rpt // zr
  nsb = ncw // SB
  niter = ncw // (2 * SB)
  mesh = plsc.VectorSubcoreMesh(core_axis_name="c", subcore_axis_name="s")

  def body(p_hbm, src_hbm, dst_hbm, out_hbm,
           sbuf0, sbuf1, dbuf0, dbuf1, rows0, rows1, zbuf, acc,
           gsem0, gsem1, isem0, isem1, zsem):
    c = lax.axis_index("c")
    s = lax.axis_index("s")
    wid = s * NC + c
    z16 = jnp.zeros((LANES,), jnp.float32)
    sbufs = (sbuf0, sbuf1)
    dbufs = (dbuf0, dbuf1)
    rows = (rows0, rows1)
    gsems = (gsem0, gsem1)
    isems = (isem0, isem1)

    def zfill(i, carry):
      for j in range(d // LANES):
        zbuf[i, pl.ds(j * LANES, LANES)] = z16
      return carry

    lax.fori_loop(0, zr, zfill, 0)
    base_row = s * rpt
    for k in range(zrep):
      pltpu.async_copy(zbuf, acc.at[pl.ds(base_row + k * zr, zr), :], zsem)
    for k in range(zrep):
      pltpu.make_async_copy(zbuf, acc.at[pl.ds(base_row + k * zr, zr), :],
                            zsem).wait()

    ib = wid * ncw

    def load_sb(t, b, sem):
      pltpu.async_copy(src_hbm.at[pl.ds(ib + t * SB, SB), :], sbufs[b], sem)
      pltpu.async_copy(dst_hbm.at[pl.ds(ib + t * SB, SB), :], dbufs[b], sem)

    def wait_sb(t, b):
      pltpu.make_async_copy(src_hbm.at[pl.ds(ib + t * SB, SB), :],
                            sbufs[b], isems[b]).wait()
      pltpu.make_async_copy(dst_hbm.at[pl.ds(ib + t * SB, SB), :],
                            dbufs[b], isems[b]).wait()

    h0, h1l = (CH + 1) // 2, CH // 2

    def gather_chunk(b, off, rbuf, sem):
      pltpu.async_copy(p_hbm.at[sbufs[b].at[off, pl.ds(0, h0)]],
                       rbuf.at[pl.ds(0, h0), :], sem)
      pltpu.async_copy(p_hbm.at[sbufs[b].at[off, pl.ds(h0, h1l)]],
                       rbuf.at[pl.ds(h0, h1l), :], sem)

    def wait_chunk(b, off, rbuf, sem):
      pltpu.make_async_copy(p_hbm.at[sbufs[b].at[off, pl.ds(0, h0)]],
                            rbuf.at[pl.ds(0, h0), :], sem).wait()
      pltpu.make_async_copy(p_hbm.at[sbufs[b].at[off, pl.ds(h0, h1l)]],
                            rbuf.at[pl.ds(h0, h1l), :], sem).wait()

    pltpu.sync_copy(src_hbm.at[pl.ds(ib, SB), :], sbuf0)
    pltpu.sync_copy(dst_hbm.at[pl.ds(ib, SB), :], dbuf0)
    load_sb(1, 1, isem1)
    plsc.subcore_barrier()
    gather_chunk(0, 0, rows0, gsem0)

    def iter16(i, carry):
      for h in range(2 * SB):
        j = 2 * SB * i + h
        sb_h, off = divmod(h, SB)
        rp = h % 2
        if h < 2 * SB - 1:
          nsb_h, noff = divmod(h + 1, SB)
          if noff == 0:
            wait_sb(2 * i + 1, 1)
          gather_chunk(nsb_h, noff, rows[(h + 1) % 2], gsems[(h + 1) % 2])
        else:
          @pl.when(j + 1 < ncw)
          def _():
            wait_sb(2 * i + 2, 0)
            gather_chunk(0, 0, rows0, gsem0)
        wait_chunk(sb_h, off, rows[rp], gsems[rp])
        pltpu.sync_copy(rows[rp], acc.at[dbufs[sb_h].at[off]], add=True)
        if h == SB - 1:
          @pl.when(2 * i + 2 < nsb)
          def _():
            load_sb(2 * i + 2, 0, isem0)
        if h == 2 * SB - 1:
          @pl.when(2 * i + 3 < nsb)
          def _():
            load_sb(2 * i + 3, 1, isem1)
      return carry

    lax.fori_loop(0, niter, iter16, 0)
    plsc.subcore_barrier()
    pltpu.sync_copy(acc.at[pl.ds(base_row, rpt), :],
                    out_hbm.at[c, pl.ds(base_row, rpt), :])

  return pl.kernel(
      body,
      out_type=jax.ShapeDtypeStruct((NC, n_acc, d), jnp.float32),
      mesh=mesh,
      scratch_types=[
          pltpu.VMEM((SB, CH), jnp.int32),
          pltpu.VMEM((SB, CH), jnp.int32),
          pltpu.VMEM((SB, CH), jnp.int32),
          pltpu.VMEM((SB, CH), jnp.int32),
          pltpu.VMEM((CH, d), jnp.float32),
          pltpu.VMEM((CH, d), jnp.float32),
          pltpu.VMEM((zr, d), jnp.float32),
          pltpu.VMEM_SHARED((n_acc, d), jnp.float32),
          pltpu.SemaphoreType.DMA,
          pltpu.SemaphoreType.DMA,
          pltpu.SemaphoreType.DMA,
          pltpu.SemaphoreType.DMA,
          pltpu.SemaphoreType.DMA,
      ],
  )


def _tc_layer1(x, w1, dega, rb=2000):
  n, d = x.shape

  def body(x_ref, w_ref, d_ref, p_ref, g_ref):
    deg = d_ref[0] + d_ref[1] + 1.0
    g = lax.rsqrt(deg)
    h = jnp.dot(x_ref[...], w_ref[...], preferred_element_type=jnp.float32)
    p_ref[...] = h * g
    g_ref[...] = g

  return pl.pallas_call(
      body,
      grid=(n // rb,),
      in_specs=[
          pl.BlockSpec((rb, d), lambda i: (i, 0)),
          pl.BlockSpec((d, d), lambda i: (0, 0)),
          pl.BlockSpec((NC, rb, 1), lambda i: (0, i, 0)),
      ],
      out_specs=[
          pl.BlockSpec((rb, d), lambda i: (i, 0)),
          pl.BlockSpec((rb, 1), lambda i: (i, 0)),
      ],
      out_shape=[
          jax.ShapeDtypeStruct((n, d), jnp.float32),
          jax.ShapeDtypeStruct((n, 1), jnp.float32),
      ],
  )(x, w1, dega)


def _tc_layer2(acc1, p1, g, b1, w2, rb=2000):
  n, d = p1.shape

  def body(a_ref, p_ref, g_ref, b_ref, w_ref, p2_ref):
    g_blk = g_ref[...]
    s = g_blk * (a_ref[0] + a_ref[1] + p_ref[...]) + b_ref[...]
    h = jnp.maximum(s, 0.0)
    h2 = jnp.dot(h, w_ref[...], preferred_element_type=jnp.float32)
    p2_ref[...] = h2 * g_blk

  return pl.pallas_call(
      body,
      grid=(n // rb,),
      in_specs=[
          pl.BlockSpec((NC, rb, d), lambda i: (0, i, 0)),
          pl.BlockSpec((rb, d), lambda i: (i, 0)),
          pl.BlockSpec((rb, 1), lambda i: (i, 0)),
          pl.BlockSpec((1, d), lambda i: (0, 0)),
          pl.BlockSpec((d, d), lambda i: (0, 0)),
      ],
      out_specs=pl.BlockSpec((rb, d), lambda i: (i, 0)),
      out_shape=jax.ShapeDtypeStruct((n, d), jnp.float32),
  )(acc1, p1, g, b1, w2)


def _tc_layer3(acc2, p2, g, b2, rb=2000):
  n, d = p2.shape

  def body(a_ref, p_ref, g_ref, b_ref, out_ref):
    out_ref[...] = (g_ref[...] * (a_ref[0] + a_ref[1] + p_ref[...])
                    + b_ref[...])

  return pl.pallas_call(
      body,
      grid=(n // rb,),
      in_specs=[
          pl.BlockSpec((NC, rb, d), lambda i: (0, i, 0)),
          pl.BlockSpec((rb, d), lambda i: (i, 0)),
          pl.BlockSpec((rb, 1), lambda i: (i, 0)),
          pl.BlockSpec((1, d), lambda i: (0, 0)),
      ],
      out_specs=pl.BlockSpec((rb, d), lambda i: (i, 0)),
      out_shape=jax.ShapeDtypeStruct((n, d), jnp.float32),
  )(acc2, p2, g, b2)


def kernel(x, edge_index, W1, b1, W2, b2):
  n, d = x.shape
  e = edge_index.shape[1]
  assert d % LANES == 0

  ept = ((n + NS - 1) // NS + 15) // 16 * 16
  n_pad = ept * NS

  quant = NW * CH * 2 * SB
  e_pad = (e + quant - 1) // quant * quant
  ncw = e_pad // (NW * CH)
  ei = edge_index.astype(jnp.int32)
  pad_src = jnp.zeros((e_pad - e,), jnp.int32)
  pad_dst = jnp.full((e_pad - e,), n_pad - 1, jnp.int32)
  src = jnp.concatenate([ei[0], pad_src]).reshape(e_pad // CH, CH)
  dst = jnp.concatenate([ei[1], pad_dst]).reshape(e_pad // CH, CH)

  deg2 = _degree_kernel(n_pad, ncw)(dst).reshape(NC, n_pad)
  dega = deg2[:, :n].reshape(NC, n, 1)

  p1, g = _tc_layer1(x, W1, dega)
  acc1 = _edge_agg_kernel(n_pad, d, ncw)(p1, src, dst)
  p2 = _tc_layer2(acc1, p1, g, b1.reshape(1, d), W2)
  acc2 = _edge_agg_kernel(n_pad, d, ncw)(p2, src, dst)
  return _tc_layer3(acc2, p2, g, b2.reshape(1, d))


# speedup vs baseline: 3.6200x; 3.6200x over previous
import functools

import jax
import jax.numpy as jnp
from jax import lax
from jax.experimental import pallas as pl
from jax.experimental.pallas import tpu as pltpu
from jax.experimental.pallas import tpu_sc as plsc

NC = 2
NS = 16
NW = NC * NS
CH = 128
LANES = 16


def _degree_kernel(n_pad, n_chunks_per_worker):
  ept = n_pad // NS
  ones_len = ((CH + LANES - 1) // LANES) * LANES
  mesh = plsc.VectorSubcoreMesh(core_axis_name="c", subcore_axis_name="s")

  def body(dst_hbm, out_hbm, dstv, ones_v, zv, acc):
    c = lax.axis_index("c")
    s = lax.axis_index("s")
    wid = s * NC + c
    one16 = jnp.ones((LANES,), jnp.float32)
    z16 = jnp.zeros((LANES,), jnp.float32)
    for j in range(ones_len // LANES):
      ones_v[pl.ds(j * LANES, LANES)] = one16
    for j in range(ept // LANES):
      zv[pl.ds(j * LANES, LANES)] = z16
    pltpu.sync_copy(zv, acc.at[pl.ds(s * ept, ept)])
    ib = wid * n_chunks_per_worker
    pltpu.sync_copy(dst_hbm.at[pl.ds(ib, n_chunks_per_worker), :], dstv)
    plsc.subcore_barrier()

    def chunk(j, carry):
      pltpu.sync_copy(ones_v.at[pl.ds(0, CH)], acc.at[dstv.at[j]], add=True)
      return carry

    lax.fori_loop(0, n_chunks_per_worker, chunk, 0)
    plsc.subcore_barrier()
    pltpu.sync_copy(acc.at[pl.ds(s * ept, ept)],
                    out_hbm.at[pl.ds(c * n_pad + s * ept, ept)])

  return pl.kernel(
      body,
      out_type=jax.ShapeDtypeStruct((NC * n_pad,), jnp.float32),
      mesh=mesh,
      scratch_types=[
          pltpu.VMEM((n_chunks_per_worker, CH), jnp.int32),
          pltpu.VMEM((ones_len,), jnp.float32),
          pltpu.VMEM((ept,), jnp.float32),
          pltpu.VMEM_SHARED((n_pad,), jnp.float32),
      ],
  )


SB = 8


def _edge_agg_kernel(n_acc, d, ncw):
  rpt = n_acc // NS
  zr = 64
  zrep = rpt // zr
  nsb = ncw // SB
  niter = ncw // (2 * SB)
  mesh = plsc.VectorSubcoreMesh(core_axis_name="c", subcore_axis_name="s")

  def body(p_hbm, src_hbm, dst_hbm, out_hbm,
           sbuf0, sbuf1, dbuf0, dbuf1, rows0, rows1, zbuf, acc,
           gsem0, gsem1, isem0, isem1, zsem):
    c = lax.axis_index("c")
    s = lax.axis_index("s")
    wid = s * NC + c
    z16 = jnp.zeros((LANES,), jnp.float32)
    sbufs = (sbuf0, sbuf1)
    dbufs = (dbuf0, dbuf1)
    rows = (rows0, rows1)
    gsems = (gsem0, gsem1)
    isems = (isem0, isem1)

    def zfill(i, carry):
      for j in range(d // LANES):
        zbuf[i, pl.ds(j * LANES, LANES)] = z16
      return carry

    lax.fori_loop(0, zr, zfill, 0)
    base_row = s * rpt
    for k in range(zrep):
      pltpu.async_copy(zbuf, acc.at[pl.ds(base_row + k * zr, zr), :], zsem)
    for k in range(zrep):
      pltpu.make_async_copy(zbuf, acc.at[pl.ds(base_row + k * zr, zr), :],
                            zsem).wait()

    ib = wid * ncw

    def load_sb(t, b, sem):
      pltpu.async_copy(src_hbm.at[pl.ds(ib + t * SB, SB), :], sbufs[b], sem)
      pltpu.async_copy(dst_hbm.at[pl.ds(ib + t * SB, SB), :], dbufs[b], sem)

    def wait_sb(t, b):
      pltpu.make_async_copy(src_hbm.at[pl.ds(ib + t * SB, SB), :],
                            sbufs[b], isems[b]).wait()
      pltpu.make_async_copy(dst_hbm.at[pl.ds(ib + t * SB, SB), :],
                            dbufs[b], isems[b]).wait()

    h0, h1l = (CH + 1) // 2, CH // 2

    def gather_chunk(b, off, rbuf, sem):
      pltpu.async_copy(p_hbm.at[sbufs[b].at[off, pl.ds(0, h0)]],
                       rbuf.at[pl.ds(0, h0), :], sem)
      pltpu.async_copy(p_hbm.at[sbufs[b].at[off, pl.ds(h0, h1l)]],
                       rbuf.at[pl.ds(h0, h1l), :], sem)

    def wait_chunk(b, off, rbuf, sem):
      pltpu.make_async_copy(p_hbm.at[sbufs[b].at[off, pl.ds(0, h0)]],
                            rbuf.at[pl.ds(0, h0), :], sem).wait()
      pltpu.make_async_copy(p_hbm.at[sbufs[b].at[off, pl.ds(h0, h1l)]],
                            rbuf.at[pl.ds(h0, h1l), :], sem).wait()

    pltpu.sync_copy(src_hbm.at[pl.ds(ib, SB), :], sbuf0)
    pltpu.sync_copy(dst_hbm.at[pl.ds(ib, SB), :], dbuf0)
    load_sb(1, 1, isem1)
    plsc.subcore_barrier()
    gather_chunk(0, 0, rows0, gsem0)

    def iter16(i, carry):
      for h in range(2 * SB):
        j = 2 * SB * i + h
        sb_h, off = divmod(h, SB)
        rp = h % 2
        if h < 2 * SB - 1:
          nsb_h, noff = divmod(h + 1, SB)
          if noff == 0:
            wait_sb(2 * i + 1, 1)
          gather_chunk(nsb_h, noff, rows[(h + 1) % 2], gsems[(h + 1) % 2])
        else:
          @pl.when(j + 1 < ncw)
          def _():
            wait_sb(2 * i + 2, 0)
            gather_chunk(0, 0, rows0, gsem0)
        wait_chunk(sb_h, off, rows[rp], gsems[rp])
        pltpu.sync_copy(rows[rp], acc.at[dbufs[sb_h].at[off]], add=True)
        if h == SB - 1:
          @pl.when(2 * i + 2 < nsb)
          def _():
            load_sb(2 * i + 2, 0, isem0)
        if h == 2 * SB - 1:
          @pl.when(2 * i + 3 < nsb)
          def _():
            load_sb(2 * i + 3, 1, isem1)
      return carry

    lax.fori_loop(0, niter, iter16, 0)
    plsc.subcore_barrier()
    pltpu.sync_copy(acc.at[pl.ds(base_row, rpt), :],
                    out_hbm.at[c, pl.ds(base_row, rpt), :])

  return pl.kernel(
      body,
      out_type=jax.ShapeDtypeStruct((NC, n_acc, d), jnp.float32),
      mesh=mesh,
      scratch_types=[
          pltpu.VMEM((SB, CH), jnp.int32),
          pltpu.VMEM((SB, CH), jnp.int32),
          pltpu.VMEM((SB, CH), jnp.int32),
          pltpu.VMEM((SB, CH), jnp.int32),
          pltpu.VMEM((CH, d), jnp.float32),
          pltpu.VMEM((CH, d), jnp.float32),
          pltpu.VMEM((zr, d), jnp.float32),
          pltpu.VMEM_SHARED((n_acc, d), jnp.float32),
          pltpu.SemaphoreType.DMA,
          pltpu.SemaphoreType.DMA,
          pltpu.SemaphoreType.DMA,
          pltpu.SemaphoreType.DMA,
          pltpu.SemaphoreType.DMA,
      ],
  )


def _tc_layer1(x, w1, dega, rb=2000):
  n, d = x.shape

  def body(x_ref, w_ref, d_ref, p_ref, g_ref):
    deg = d_ref[0] + d_ref[1] + 1.0
    g = lax.rsqrt(deg)
    h = jnp.dot(x_ref[...], w_ref[...], preferred_element_type=jnp.float32)
    p_ref[...] = h * g
    g_ref[...] = g

  return pl.pallas_call(
      body,
      grid=(n // rb,),
      in_specs=[
          pl.BlockSpec((rb, d), lambda i: (i, 0)),
          pl.BlockSpec((d, d), lambda i: (0, 0)),
          pl.BlockSpec((NC, rb, 1), lambda i: (0, i, 0)),
      ],
      out_specs=[
          pl.BlockSpec((rb, d), lambda i: (i, 0)),
          pl.BlockSpec((rb, 1), lambda i: (i, 0)),
      ],
      out_shape=[
          jax.ShapeDtypeStruct((n, d), jnp.float32),
          jax.ShapeDtypeStruct((n, 1), jnp.float32),
      ],
  )(x, w1, dega)


def _tc_layer2(acc1, p1, g, b1, w2, rb=2000):
  n, d = p1.shape

  def body(a_ref, p_ref, g_ref, b_ref, w_ref, p2_ref):
    g_blk = g_ref[...]
    s = g_blk * (a_ref[0] + a_ref[1] + p_ref[...]) + b_ref[...]
    h = jnp.maximum(s, 0.0)
    h2 = jnp.dot(h, w_ref[...], preferred_element_type=jnp.float32)
    p2_ref[...] = h2 * g_blk

  return pl.pallas_call(
      body,
      grid=(n // rb,),
      in_specs=[
          pl.BlockSpec((NC, rb, d), lambda i: (0, i, 0)),
          pl.BlockSpec((rb, d), lambda i: (i, 0)),
          pl.BlockSpec((rb, 1), lambda i: (i, 0)),
          pl.BlockSpec((1, d), lambda i: (0, 0)),
          pl.BlockSpec((d, d), lambda i: (0, 0)),
      ],
      out_specs=pl.BlockSpec((rb, d), lambda i: (i, 0)),
      out_shape=jax.ShapeDtypeStruct((n, d), jnp.float32),
  )(acc1, p1, g, b1, w2)


def _tc_layer3(acc2, p2, g, b2, rb=2000):
  n, d = p2.shape

  def body(a_ref, p_ref, g_ref, b_ref, out_ref):
    out_ref[...] = (g_ref[...] * (a_ref[0] + a_ref[1] + p_ref[...])
                    + b_ref[...])

  return pl.pallas_call(
      body,
      grid=(n // rb,),
      in_specs=[
          pl.BlockSpec((NC, rb, d), lambda i: (0, i, 0)),
          pl.BlockSpec((rb, d), lambda i: (i, 0)),
          pl.BlockSpec((rb, 1), lambda i: (i, 0)),
          pl.BlockSpec((1, d), lambda i: (0, 0)),
      ],
      out_specs=pl.BlockSpec((rb, d), lambda i: (i, 0)),
      out_shape=jax.ShapeDtypeStruct((n, d), jnp.float32),
  )(acc2, p2, g, b2)


def kernel(x, edge_index, W1, b1, W2, b2):
  n, d = x.shape
  e = edge_index.shape[1]
  assert d % LANES == 0

  ept = ((n + NS - 1) // NS + 15) // 16 * 16
  n_pad = ept * NS

  quant = NW * CH * 2 * SB
  e_pad = (e + quant - 1) // quant * quant
  ncw = e_pad // (NW * CH)
  ei = edge_index.astype(jnp.int32)
  npad_ix = jnp.arange(e_pad - e, dtype=jnp.int32)
  pad_src = npad_ix % n
  pad_dst = n + npad_ix % (n_pad - n)
  src = jnp.concatenate([ei[0], pad_src]).reshape(e_pad // CH, CH)
  dst = jnp.concatenate([ei[1], pad_dst]).reshape(e_pad // CH, CH)

  deg2 = _degree_kernel(n_pad, ncw)(dst).reshape(NC, n_pad)
  dega = deg2[:, :n].reshape(NC, n, 1)

  p1, g = _tc_layer1(x, W1, dega)
  acc1 = _edge_agg_kernel(n_pad, d, ncw)(p1, src, dst)
  p2 = _tc_layer2(acc1, p1, g, b1.reshape(1, d), W2)
  acc2 = _edge_agg_kernel(n_pad, d, ncw)(p2, src, dst)
  return _tc_layer3(acc2, p2, g, b2.reshape(1, d))


# speedup vs baseline: 3.6697x; 1.0137x over previous
import functools

import jax
import jax.numpy as jnp
from jax import lax
from jax.experimental import pallas as pl
from jax.experimental.pallas import tpu as pltpu
from jax.experimental.pallas import tpu_sc as plsc

NC = 2
NS = 16
NW = NC * NS
CH = 128
LANES = 16


def _degree_kernel(n_pad, n_chunks_per_worker):
  ept = n_pad // NS
  ones_len = ((CH + LANES - 1) // LANES) * LANES
  mesh = plsc.VectorSubcoreMesh(core_axis_name="c", subcore_axis_name="s")

  def body(dst_hbm, out_hbm, dstv, ones_v, zv, acc, dsem):
    c = lax.axis_index("c")
    s = lax.axis_index("s")
    wid = s * NC + c
    one16 = jnp.ones((LANES,), jnp.float32)
    z16 = jnp.zeros((LANES,), jnp.float32)
    for j in range(ones_len // LANES):
      ones_v[pl.ds(j * LANES, LANES)] = one16
    for j in range(ept // LANES):
      zv[pl.ds(j * LANES, LANES)] = z16
    pltpu.sync_copy(zv, acc.at[pl.ds(s * ept, ept)])
    ib = wid * n_chunks_per_worker
    pltpu.sync_copy(dst_hbm.at[pl.ds(ib, n_chunks_per_worker), :], dstv)
    plsc.subcore_barrier()

    kb = 16

    def grp(t, carry):
      for k in range(kb):
        pltpu.async_copy(ones_v.at[pl.ds(0, CH)],
                         acc.at[dstv.at[t * kb + k]], dsem, add=True)
      for k in range(kb):
        pltpu.make_async_copy(ones_v.at[pl.ds(0, CH)],
                              acc.at[dstv.at[t * kb + k]], dsem).wait()
      return carry

    lax.fori_loop(0, n_chunks_per_worker // kb, grp, 0)
    plsc.subcore_barrier()
    pltpu.sync_copy(acc.at[pl.ds(s * ept, ept)],
                    out_hbm.at[pl.ds(c * n_pad + s * ept, ept)])

  return pl.kernel(
      body,
      out_type=jax.ShapeDtypeStruct((NC * n_pad,), jnp.float32),
      mesh=mesh,
      scratch_types=[
          pltpu.VMEM((n_chunks_per_worker, CH), jnp.int32),
          pltpu.VMEM((ones_len,), jnp.float32),
          pltpu.VMEM((ept,), jnp.float32),
          pltpu.VMEM_SHARED((n_pad,), jnp.float32),
          pltpu.SemaphoreType.DMA,
      ],
  )


SB = 8


def _edge_agg_kernel(n_acc, d, ncw):
  rpt = n_acc // NS
  zr = 64
  zrep = rpt // zr
  nsb = ncw // SB
  niter = ncw // (2 * SB)
  mesh = plsc.VectorSubcoreMesh(core_axis_name="c", subcore_axis_name="s")

  def body(p_hbm, src_hbm, dst_hbm, out_hbm,
           sbuf0, sbuf1, dbuf0, dbuf1, rows0, rows1, zbuf, acc,
           gsem0, gsem1, isem0, isem1, zsem):
    c = lax.axis_index("c")
    s = lax.axis_index("s")
    wid = s * NC + c
    z16 = jnp.zeros((LANES,), jnp.float32)
    sbufs = (sbuf0, sbuf1)
    dbufs = (dbuf0, dbuf1)
    rows = (rows0, rows1)
    gsems = (gsem0, gsem1)
    isems = (isem0, isem1)

    def zfill(i, carry):
      for j in range(d // LANES):
        zbuf[i, pl.ds(j * LANES, LANES)] = z16
      return carry

    lax.fori_loop(0, zr, zfill, 0)
    base_row = s * rpt
    for k in range(zrep):
      pltpu.async_copy(zbuf, acc.at[pl.ds(base_row + k * zr, zr), :], zsem)
    for k in range(zrep):
      pltpu.make_async_copy(zbuf, acc.at[pl.ds(base_row + k * zr, zr), :],
                            zsem).wait()

    ib = wid * ncw

    def load_sb(t, b, sem):
      pltpu.async_copy(src_hbm.at[pl.ds(ib + t * SB, SB), :], sbufs[b], sem)
      pltpu.async_copy(dst_hbm.at[pl.ds(ib + t * SB, SB), :], dbufs[b], sem)

    def wait_sb(t, b):
      pltpu.make_async_copy(src_hbm.at[pl.ds(ib + t * SB, SB), :],
                            sbufs[b], isems[b]).wait()
      pltpu.make_async_copy(dst_hbm.at[pl.ds(ib + t * SB, SB), :],
                            dbufs[b], isems[b]).wait()

    h0, h1l = (CH + 1) // 2, CH // 2

    def gather_chunk(b, off, rbuf, sem):
      pltpu.async_copy(p_hbm.at[sbufs[b].at[off, pl.ds(0, h0)]],
                       rbuf.at[pl.ds(0, h0), :], sem)
      pltpu.async_copy(p_hbm.at[sbufs[b].at[off, pl.ds(h0, h1l)]],
                       rbuf.at[pl.ds(h0, h1l), :], sem)

    def wait_chunk(b, off, rbuf, sem):
      pltpu.make_async_copy(p_hbm.at[sbufs[b].at[off, pl.ds(0, h0)]],
                            rbuf.at[pl.ds(0, h0), :], sem).wait()
      pltpu.make_async_copy(p_hbm.at[sbufs[b].at[off, pl.ds(h0, h1l)]],
                            rbuf.at[pl.ds(h0, h1l), :], sem).wait()

    pltpu.sync_copy(src_hbm.at[pl.ds(ib, SB), :], sbuf0)
    pltpu.sync_copy(dst_hbm.at[pl.ds(ib, SB), :], dbuf0)
    load_sb(1, 1, isem1)
    plsc.subcore_barrier()
    gather_chunk(0, 0, rows0, gsem0)

    def iter16(i, carry):
      for h in range(2 * SB):
        j = 2 * SB * i + h
        sb_h, off = divmod(h, SB)
        rp = h % 2
        if h < 2 * SB - 1:
          nsb_h, noff = divmod(h + 1, SB)
          if noff == 0:
            wait_sb(2 * i + 1, 1)
          gather_chunk(nsb_h, noff, rows[(h + 1) % 2], gsems[(h + 1) % 2])
        else:
          @pl.when(j + 1 < ncw)
          def _():
            wait_sb(2 * i + 2, 0)
            gather_chunk(0, 0, rows0, gsem0)
        wait_chunk(sb_h, off, rows[rp], gsems[rp])
        pltpu.sync_copy(rows[rp], acc.at[dbufs[sb_h].at[off]], add=True)
        if h == SB - 1:
          @pl.when(2 * i + 2 < nsb)
          def _():
            load_sb(2 * i + 2, 0, isem0)
        if h == 2 * SB - 1:
          @pl.when(2 * i + 3 < nsb)
          def _():
            load_sb(2 * i + 3, 1, isem1)
      return carry

    lax.fori_loop(0, niter, iter16, 0)
    plsc.subcore_barrier()
    pltpu.sync_copy(acc.at[pl.ds(base_row, rpt), :],
                    out_hbm.at[c, pl.ds(base_row, rpt), :])

  return pl.kernel(
      body,
      out_type=jax.ShapeDtypeStruct((NC, n_acc, d), jnp.float32),
      mesh=mesh,
      scratch_types=[
          pltpu.VMEM((SB, CH), jnp.int32),
          pltpu.VMEM((SB, CH), jnp.int32),
          pltpu.VMEM((SB, CH), jnp.int32),
          pltpu.VMEM((SB, CH), jnp.int32),
          pltpu.VMEM((CH, d), jnp.float32),
          pltpu.VMEM((CH, d), jnp.float32),
          pltpu.VMEM((zr, d), jnp.float32),
          pltpu.VMEM_SHARED((n_acc, d), jnp.float32),
          pltpu.SemaphoreType.DMA,
          pltpu.SemaphoreType.DMA,
          pltpu.SemaphoreType.DMA,
          pltpu.SemaphoreType.DMA,
          pltpu.SemaphoreType.DMA,
      ],
  )


def _tc_layer1(x, w1, dega, rb=2000):
  n, d = x.shape

  def body(x_ref, w_ref, d_ref, p_ref, g_ref):
    deg = d_ref[0] + d_ref[1] + 1.0
    g = lax.rsqrt(deg)
    h = jnp.dot(x_ref[...], w_ref[...], preferred_element_type=jnp.float32)
    p_ref[...] = h * g
    g_ref[...] = g

  return pl.pallas_call(
      body,
      grid=(n // rb,),
      in_specs=[
          pl.BlockSpec((rb, d), lambda i: (i, 0)),
          pl.BlockSpec((d, d), lambda i: (0, 0)),
          pl.BlockSpec((NC, rb, 1), lambda i: (0, i, 0)),
      ],
      out_specs=[
          pl.BlockSpec((rb, d), lambda i: (i, 0)),
          pl.BlockSpec((rb, 1), lambda i: (i, 0)),
      ],
      out_shape=[
          jax.ShapeDtypeStruct((n, d), jnp.float32),
          jax.ShapeDtypeStruct((n, 1), jnp.float32),
      ],
  )(x, w1, dega)


def _tc_layer2(acc1, p1, g, b1, w2, rb=2000):
  n, d = p1.shape

  def body(a_ref, p_ref, g_ref, b_ref, w_ref, p2_ref):
    g_blk = g_ref[...]
    s = g_blk * (a_ref[0] + a_ref[1] + p_ref[...]) + b_ref[...]
    h = jnp.maximum(s, 0.0)
    h2 = jnp.dot(h, w_ref[...], preferred_element_type=jnp.float32)
    p2_ref[...] = h2 * g_blk

  return pl.pallas_call(
      body,
      grid=(n // rb,),
      in_specs=[
          pl.BlockSpec((NC, rb, d), lambda i: (0, i, 0)),
          pl.BlockSpec((rb, d), lambda i: (i, 0)),
          pl.BlockSpec((rb, 1), lambda i: (i, 0)),
          pl.BlockSpec((1, d), lambda i: (0, 0)),
          pl.BlockSpec((d, d), lambda i: (0, 0)),
      ],
      out_specs=pl.BlockSpec((rb, d), lambda i: (i, 0)),
      out_shape=jax.ShapeDtypeStruct((n, d), jnp.float32),
  )(acc1, p1, g, b1, w2)


def _tc_layer3(acc2, p2, g, b2, rb=2000):
  n, d = p2.shape

  def body(a_ref, p_ref, g_ref, b_ref, out_ref):
    out_ref[...] = (g_ref[...] * (a_ref[0] + a_ref[1] + p_ref[...])
                    + b_ref[...])

  return pl.pallas_call(
      body,
      grid=(n // rb,),
      in_specs=[
          pl.BlockSpec((NC, rb, d), lambda i: (0, i, 0)),
          pl.BlockSpec((rb, d), lambda i: (i, 0)),
          pl.BlockSpec((rb, 1), lambda i: (i, 0)),
          pl.BlockSpec((1, d), lambda i: (0, 0)),
      ],
      out_specs=pl.BlockSpec((rb, d), lambda i: (i, 0)),
      out_shape=jax.ShapeDtypeStruct((n, d), jnp.float32),
  )(acc2, p2, g, b2)


def kernel(x, edge_index, W1, b1, W2, b2):
  n, d = x.shape
  e = edge_index.shape[1]
  assert d % LANES == 0

  ept = ((n + NS - 1) // NS + 15) // 16 * 16
  n_pad = ept * NS

  quant = NW * CH * 2 * SB
  e_pad = (e + quant - 1) // quant * quant
  ncw = e_pad // (NW * CH)
  ei = edge_index.astype(jnp.int32)
  npad_ix = jnp.arange(e_pad - e, dtype=jnp.int32)
  pad_src = npad_ix % n
  pad_dst = n + npad_ix % (n_pad - n)
  src = jnp.concatenate([ei[0], pad_src]).reshape(e_pad // CH, CH)
  dst = jnp.concatenate([ei[1], pad_dst]).reshape(e_pad // CH, CH)

  deg2 = _degree_kernel(n_pad, ncw)(dst).reshape(NC, n_pad)
  dega = deg2[:, :n].reshape(NC, n, 1)

  p1, g = _tc_layer1(x, W1, dega)
  acc1 = _edge_agg_kernel(n_pad, d, ncw)(p1, src, dst)
  p2 = _tc_layer2(acc1, p1, g, b1.reshape(1, d), W2)
  acc2 = _edge_agg_kernel(n_pad, d, ncw)(p2, src, dst)
  return _tc_layer3(acc2, p2, g, b2.reshape(1, d))
